# Initial kernel scaffold; baseline (speedup 1.0000x reference)
import jax
import jax.numpy as jnp
from jax.experimental import pallas as pl


def kernel(x, edge_index, W1, b1, W2, b2, Wl, bl):
    raise NotImplementedError("write your pallas kernel here")


# speedup vs baseline: 10.4199x; 10.4199x over previous
import functools

import jax
import jax.numpy as jnp
from jax import lax
from jax.experimental import pallas as pl
from jax.experimental.pallas import tpu as pltpu
from jax.experimental.pallas import tpu_sc as plsc

NC = 2
NS = 16
NW = NC * NS
CHUNK = 128
DEGW = 16


def _mesh():
    return plsc.VectorSubcoreMesh(core_axis_name="c", subcore_axis_name="s",
                                  num_cores=NC, num_subcores=NS)


def _make_deg(npad, epad):
    ept = epad // NW
    nchunks = ept // CHUNK
    spt = npad // NS

    @functools.partial(
        pl.kernel,
        out_type=jax.ShapeDtypeStruct((NC * npad,), jnp.float32),
        mesh=_mesh(),
        compiler_params=pltpu.CompilerParams(needs_layout_passes=False),
        scratch_types=[
            pltpu.VMEM_SHARED((NS * npad,), jnp.float32),
            pltpu.VMEM((npad,), jnp.float32),
            pltpu.VMEM((npad,), jnp.float32),
            pltpu.VMEM((spt,), jnp.float32),
            pltpu.VMEM((CHUNK,), jnp.int32),
        ],
    )
    def deg_kernel(dst_hbm, zeros_hbm, out_hbm, shared, hist, hbuf, rbuf,
                   didx):
        c = lax.axis_index("c")
        s = lax.axis_index("s")
        wid = c * NS + s
        pltpu.sync_copy(zeros_hbm, hist)
        ebase = wid * ept
        ones = jnp.ones((16,), jnp.float32)

        def body(i, carry):
            b = ebase + i * CHUNK
            pltpu.sync_copy(dst_hbm.at[pl.ds(b, CHUNK)], didx)
            for j in range(CHUNK // 16):
                idx = didx[pl.ds(j * 16, 16)]
                plsc.addupdate_scatter(hist, [idx], ones)
            return carry

        lax.fori_loop(0, nchunks, body, 0)
        pltpu.sync_copy(hist, shared.at[pl.ds(s * npad, npad)])
        plsc.subcore_barrier()
        for t in range(NS):
            pltpu.sync_copy(shared.at[pl.ds(t * npad + s * spt, spt)],
                            hbuf.at[pl.ds(t * spt, spt)])

        def rbody(k, carry):
            v = hbuf[pl.ds(k * 16, 16)]
            for t in range(1, NS):
                v = v + hbuf[pl.ds(t * spt + k * 16, 16)]
            rbuf[pl.ds(k * 16, 16)] = v
            return carry

        lax.fori_loop(0, spt // 16, rbody, 0)
        pltpu.sync_copy(rbuf, out_hbm.at[pl.ds(c * npad + s * spt, spt)])

    return deg_kernel


def _make_agg(npad, epad):
    ept = epad // NW
    nchunks = ept // CHUNK
    rpt = npad // NS

    @functools.partial(
        pl.kernel,
        out_type=jax.ShapeDtypeStruct((NC * npad, 128), jnp.float32),
        mesh=_mesh(),
        scratch_types=[
            pltpu.VMEM_SHARED((npad, 128), jnp.float32),
            pltpu.VMEM((CHUNK,), jnp.int32),
            pltpu.VMEM((CHUNK,), jnp.int32),
            pltpu.VMEM((CHUNK, 128), jnp.float32),
            pltpu.SemaphoreType.DMA,
        ],
    )
    def agg_kernel(g_hbm, src_hbm, dst_hbm, zeros_hbm, out_hbm,
                   acc, sidx, didx, rows, sem):
        c = lax.axis_index("c")
        s = lax.axis_index("s")
        wid = c * NS + s
        pltpu.sync_copy(zeros_hbm, acc.at[pl.ds(s * rpt, rpt)])
        plsc.subcore_barrier()
        ebase = wid * ept

        def body(i, carry):
            b = ebase + i * CHUNK
            pltpu.sync_copy(src_hbm.at[pl.ds(b, CHUNK)], sidx)
            pltpu.sync_copy(dst_hbm.at[pl.ds(b, CHUNK)], didx)
            pltpu.async_copy(g_hbm.at[sidx], rows, sem).wait()
            pltpu.sync_copy(rows, acc.at[didx], add=True)
            return carry

        lax.fori_loop(0, nchunks, body, 0)
        plsc.subcore_barrier()
        pltpu.sync_copy(acc.at[pl.ds(s * rpt, rpt)],
                        out_hbm.at[pl.ds(c * npad + s * rpt, rpt)])

    return agg_kernel


def _tc1(x_p, W1, deg_col, bn):
    npad = x_p.shape[0]
    nb = npad // bn

    def body(x_ref, w_ref, d_ref, g_ref):
        dinv = lax.rsqrt(d_ref[...])
        g_ref[...] = jnp.dot(x_ref[...], w_ref[...],
                             preferred_element_type=jnp.float32) * dinv

    return pl.pallas_call(
        body,
        grid=(nb,),
        in_specs=[
            pl.BlockSpec((bn, 128), lambda i: (i, 0)),
            pl.BlockSpec((128, 128), lambda i: (0, 0)),
            pl.BlockSpec((bn, 1), lambda i: (i, 0)),
        ],
        out_specs=pl.BlockSpec((bn, 128), lambda i: (i, 0)),
        out_shape=jax.ShapeDtypeStruct((npad, 128), jnp.float32),
    )(x_p, W1, deg_col)


def _tc2(parts, g1, deg_col, b1, W2, bn):
    npad = g1.shape[0]
    nb = npad // bn

    def body(p0, p1, g_ref, d_ref, b_ref, w_ref, out_ref):
        dinv = lax.rsqrt(d_ref[...])
        h = jnp.maximum((p0[...] + p1[...] + g_ref[...]) * dinv + b_ref[...],
                        0.0)
        out_ref[...] = jnp.dot(h, w_ref[...],
                               preferred_element_type=jnp.float32) * dinv

    return pl.pallas_call(
        body,
        grid=(nb,),
        in_specs=[
            pl.BlockSpec((bn, 128), lambda i: (i, 0)),
            pl.BlockSpec((bn, 128), lambda i, _nb=nb: (i + _nb, 0)),
            pl.BlockSpec((bn, 128), lambda i: (i, 0)),
            pl.BlockSpec((bn, 1), lambda i: (i, 0)),
            pl.BlockSpec((1, 128), lambda i: (0, 0)),
            pl.BlockSpec((128, 128), lambda i: (0, 0)),
        ],
        out_specs=pl.BlockSpec((bn, 128), lambda i: (i, 0)),
        out_shape=jax.ShapeDtypeStruct((npad, 128), jnp.float32),
    )(parts, parts, g1, deg_col, b1, W2)


def _tc3(parts, g2, deg_col, b2, Wl, bl, bn):
    npad = g2.shape[0]
    nb = npad // bn
    cdim = Wl.shape[1]

    def body(p0, p1, g_ref, d_ref, b_ref, w_ref, bl_ref, out_ref):
        dinv = lax.rsqrt(d_ref[...])
        h = jnp.maximum((p0[...] + p1[...] + g_ref[...]) * dinv + b_ref[...],
                        0.0)
        out_ref[...] = jnp.dot(h, w_ref[...],
                               preferred_element_type=jnp.float32) + bl_ref[...]

    return pl.pallas_call(
        body,
        grid=(nb,),
        in_specs=[
            pl.BlockSpec((bn, 128), lambda i: (i, 0)),
            pl.BlockSpec((bn, 128), lambda i, _nb=nb: (i + _nb, 0)),
            pl.BlockSpec((bn, 128), lambda i: (i, 0)),
            pl.BlockSpec((bn, 1), lambda i: (i, 0)),
            pl.BlockSpec((1, 128), lambda i: (0, 0)),
            pl.BlockSpec((128, cdim), lambda i: (0, 0)),
            pl.BlockSpec((1, cdim), lambda i: (0, 0)),
        ],
        out_specs=pl.BlockSpec((bn, cdim), lambda i: (i, 0)),
        out_shape=jax.ShapeDtypeStruct((npad, cdim), jnp.float32),
    )(parts, parts, g2, deg_col, b2, Wl, bl)


def kernel(x, edge_index, W1, b1, W2, b2, Wl, bl):
    n = x.shape[0]
    e = edge_index.shape[1]
    npad = ((n + 1 + 511) // 512) * 512
    epad = -(-e // (NW * CHUNK)) * (NW * CHUNK)
    rpt = npad // NS
    bn = 1024 if npad % 1024 == 0 else 512

    src = edge_index[0].astype(jnp.int32)
    dst = edge_index[1].astype(jnp.int32)
    src_p = jnp.concatenate([src, jnp.zeros((epad - e,), jnp.int32)])
    dst_p = jnp.concatenate([dst, jnp.full((epad - e,), n, jnp.int32)])
    x_p = jnp.pad(x, ((0, npad - n), (0, 0)))

    zeros128 = jnp.zeros((rpt, 128), jnp.float32)
    zeros1d = jnp.zeros((npad,), jnp.float32)

    deg_call = _make_deg(npad, epad)
    agg_call = _make_agg(npad, epad)

    dparts = deg_call(dst_p, zeros1d)
    deg_col = (dparts[:npad] + dparts[npad:] + 1.0).reshape(npad, 1)

    g1 = _tc1(x_p, W1, deg_col, bn)
    p1 = agg_call(g1, src_p, dst_p, zeros128)
    g2 = _tc2(p1, g1, deg_col, b1.reshape(1, -1), W2, bn)
    p2 = agg_call(g2, src_p, dst_p, zeros128)
    out = _tc3(p2, g2, deg_col, b2.reshape(1, -1), Wl, bl.reshape(1, -1), bn)
    return out[:n]
